# baseline (speedup 1.0000x reference)
import functools

import jax
import jax.numpy as jnp
from jax import lax
from jax.experimental import pallas as pl
from jax.experimental.pallas import tpu as pltpu
from jax.experimental.pallas import tpu_sc as plsc

VOCAB = 1000000
D = 64
SEQ = 200
BATCH = 4096

_INFO = plsc.get_sparse_core_info()
NC, NS = _INFO.num_cores, _INFO.num_subcores
NW = NC * NS
SEQ_PER_W = BATCH // NW
SCALE = 1.0 / 8.0


def _embed_kernel(idx_hbm, table_hbm, pe_hbm, out_hbm, idx_v, rows_v, pe_v, sem, sem2):
    wid = lax.axis_index("s") * NC + lax.axis_index("c")
    pltpu.sync_copy(pe_hbm.at[pl.ds(0, SEQ)], pe_v)

    def per_seq(q, carry):
        base = (wid * SEQ_PER_W + q) * SEQ
        pltpu.sync_copy(idx_hbm.at[pl.ds(base, SEQ)], idx_v)
        cp1 = pltpu.make_async_copy(
            table_hbm.at[idx_v.at[pl.ds(0, 128)]], rows_v.at[pl.ds(0, 128)], sem)
        cp2 = pltpu.make_async_copy(
            table_hbm.at[idx_v.at[pl.ds(128, 72)]], rows_v.at[pl.ds(128, 72)], sem2)
        cp1.start()
        cp2.start()
        cp1.wait()
        cp2.wait()

        def per_row(r, carry2):
            for d in range(D // 16):
                sl = pl.ds(d * 16, 16)
                rows_v[r, sl] = rows_v[r, sl] * SCALE + pe_v[r, sl]
            return carry2

        lax.fori_loop(0, SEQ, per_row, 0, unroll=2)
        pltpu.sync_copy(rows_v, out_hbm.at[pl.ds(base, SEQ)])
        return carry

    lax.fori_loop(0, SEQ_PER_W, per_seq, 0)


@jax.jit
def kernel(input_seqs, table, pe):
    idx_flat = input_seqs.reshape(-1).astype(jnp.int32)
    run = pl.kernel(
        _embed_kernel,
        out_type=jax.ShapeDtypeStruct((BATCH * SEQ, D), jnp.float32),
        mesh=plsc.VectorSubcoreMesh(core_axis_name="c", subcore_axis_name="s"),
        compiler_params=pltpu.CompilerParams(use_tc_tiling_on_sc=False),
        scratch_types=[
            pltpu.VMEM((SEQ,), jnp.int32),
            pltpu.VMEM((SEQ, D), jnp.float32),
            pltpu.VMEM((SEQ, D), jnp.float32),
            pltpu.SemaphoreType.DMA,
            pltpu.SemaphoreType.DMA,
        ],
    )
    out = run(idx_flat, table, pe)
    return out.reshape(BATCH, SEQ, D)


# speedup vs baseline: 1.1754x; 1.1754x over previous
import jax
import jax.numpy as jnp
from jax import lax
from jax.experimental import pallas as pl
from jax.experimental.pallas import tpu as pltpu
from jax.experimental.pallas import tpu_sc as plsc

D = 64
SEQ = 200
BATCH = 4096

_INFO = plsc.get_sparse_core_info()
NC, NS = _INFO.num_cores, _INFO.num_subcores
NW = NC * NS
SEQ_PER_W = BATCH // NW
SCALE = 1.0 / 8.0
NBUF = 4
NPAIR = SEQ_PER_W // NBUF


def _embed_kernel(idx_hbm, table_hbm, pe_hbm, out_hbm,
                  idx_all, pe_v, rows, gsema, gsemb, osem):
    wid = lax.axis_index("s") * NC + lax.axis_index("c")
    w_base = wid * SEQ_PER_W * SEQ

    pltpu.sync_copy(pe_hbm.at[pl.ds(0, SEQ)], pe_v)
    pltpu.sync_copy(idx_hbm.at[pl.ds(wid * SEQ_PER_W, SEQ_PER_W)], idx_all)

    def fire_gather(q, j):
        pltpu.make_async_copy(
            table_hbm.at[idx_all.at[q, pl.ds(0, 128)]],
            rows[j].at[pl.ds(0, 128)], gsema[j]).start()
        pltpu.make_async_copy(
            table_hbm.at[idx_all.at[q, pl.ds(128, 72)]],
            rows[j].at[pl.ds(128, 72)], gsemb[j]).start()

    def wait_gather(j):
        pltpu.make_async_copy(
            table_hbm.at[idx_all.at[0, pl.ds(0, 128)]],
            rows[j].at[pl.ds(0, 128)], gsema[j]).wait()
        pltpu.make_async_copy(
            table_hbm.at[idx_all.at[0, pl.ds(128, 72)]],
            rows[j].at[pl.ds(128, 72)], gsemb[j]).wait()

    def fire_out(q, j):
        pltpu.make_async_copy(
            rows[j], out_hbm.at[pl.ds(w_base + q * SEQ, SEQ)], osem[j]).start()

    def wait_out(j):
        pltpu.make_async_copy(
            rows[j], out_hbm.at[pl.ds(w_base, SEQ)], osem[j]).wait()

    def compute(j):
        rbuf = rows[j]

        def per_row(r, carry):
            for d in range(D // 16):
                sl = pl.ds(d * 16, 16)
                rbuf[r, sl] = rbuf[r, sl] * SCALE + pe_v[r, sl]
            return carry

        lax.fori_loop(0, SEQ, per_row, 0, unroll=8)

    fire_gather(0, 0)

    def pair(p, carry):
        for j in range(NBUF):
            q = p * NBUF + j
            nxt = (j + 1) % NBUF
            if j < NBUF - 1:
                @pl.when(p >= 1)
                def _():
                    wait_out(nxt)
                fire_gather(q + 1, nxt)
            else:
                wait_out(nxt)

                @pl.when(p < NPAIR - 1)
                def _():
                    fire_gather(q + 1, nxt)
            wait_gather(j)
            compute(j)
            fire_out(q, j)
        return carry

    lax.fori_loop(0, NPAIR, pair, 0)
    for j in range(1, NBUF):
        wait_out(j)


@jax.jit
def kernel(input_seqs, table, pe):
    idx2d = input_seqs.astype(jnp.int32)
    run = pl.kernel(
        _embed_kernel,
        out_type=jax.ShapeDtypeStruct((BATCH * SEQ, D), jnp.float32),
        mesh=plsc.VectorSubcoreMesh(core_axis_name="c", subcore_axis_name="s"),
        compiler_params=pltpu.CompilerParams(use_tc_tiling_on_sc=False),
        scratch_types=[
            pltpu.VMEM((SEQ_PER_W, SEQ), jnp.int32),
            pltpu.VMEM((SEQ, D), jnp.float32),
            [pltpu.VMEM((SEQ, D), jnp.float32)] * NBUF,
            [pltpu.SemaphoreType.DMA] * NBUF,
            [pltpu.SemaphoreType.DMA] * NBUF,
            [pltpu.SemaphoreType.DMA] * NBUF,
        ],
    )
    out = run(idx2d, table, pe)
    return out.reshape(BATCH, SEQ, D)
